# baseline (speedup 1.0000x reference)
import functools

import jax
import jax.numpy as jnp
from jax import lax
from jax.experimental import pallas as pl
from jax.experimental.pallas import tpu as pltpu
from jax.experimental.pallas import tpu_sc as plsc

_NC = 2
_NS = 16
_NW = _NC * _NS
_L = 16

_BATCH = 4
_SEQ = 2048
_D = 1024
_S_PER_W = _SEQ // _NW
_CHUNK = 32
_NHALF = _S_PER_W // _CHUNK
_NCHUNK = _NHALF * _BATCH


def _body(tok_hbm, emb_hbm, pos_hbm, out_hbm,
          pbuf, ebuf0, ebuf1, idxv, gsem0, gsem1, ssem0, ssem1):
    wid = lax.axis_index("s") * _NC + lax.axis_index("c")
    s0 = wid * _S_PER_W

    for b in range(_BATCH):
        pltpu.sync_copy(tok_hbm.at[pl.ds(b * _SEQ + s0, _S_PER_W)],
                        idxv.at[pl.ds(b * _S_PER_W, _S_PER_W)])

    ebufs = (ebuf0, ebuf1)
    gsems = (gsem0, gsem1)
    ssems = (ssem0, ssem1)

    def idx_off(k):
        return (k % _BATCH) * _S_PER_W + (k // _BATCH) * _CHUNK

    def out_off(k):
        return (k % _BATCH) * _SEQ + s0 + (k // _BATCH) * _CHUNK

    gat = [None, None]
    scat = [None, None]

    pltpu.sync_copy(pos_hbm.at[pl.ds(s0, _CHUNK)], pbuf)
    gat[0] = pltpu.async_copy(
        emb_hbm.at[idxv.at[pl.ds(idx_off(0), _CHUNK)]], ebufs[0], gsems[0])

    for k in range(_NCHUNK):
        cur = k % 2
        nxt = 1 - cur
        if k + 1 < _NCHUNK:
            if scat[nxt] is not None:
                scat[nxt].wait()
            gat[nxt] = pltpu.async_copy(
                emb_hbm.at[idxv.at[pl.ds(idx_off(k + 1), _CHUNK)]],
                ebufs[nxt], gsems[nxt])
        gat[cur].wait()

        if k == _BATCH:
            pltpu.sync_copy(pos_hbm.at[pl.ds(s0 + _CHUNK, _CHUNK)], pbuf)

        ebuf = ebufs[cur]

        def row_add(r, carry):
            for j in range(_D // _L):
                sl = pl.ds(j * _L, _L)
                plsc.addupdate(ebuf.at[r, sl], pbuf[r, sl])
            return carry

        lax.fori_loop(0, _CHUNK, row_add, 0)
        scat[cur] = pltpu.async_copy(
            ebuf, out_hbm.at[pl.ds(out_off(k), _CHUNK)], ssems[cur])

    scat[0].wait()
    scat[1].wait()


_mesh = plsc.VectorSubcoreMesh(core_axis_name="c", subcore_axis_name="s")

_embed = pl.kernel(
    _body,
    out_type=jax.ShapeDtypeStruct((_BATCH * _SEQ, _D), jnp.float32),
    mesh=_mesh,
    scratch_types=[
        pltpu.VMEM((_CHUNK, _D), jnp.float32),
        pltpu.VMEM((_CHUNK, _D), jnp.float32),
        pltpu.VMEM((_CHUNK, _D), jnp.float32),
        pltpu.VMEM((_BATCH * _S_PER_W,), jnp.int32),
        pltpu.SemaphoreType.DMA,
        pltpu.SemaphoreType.DMA,
        pltpu.SemaphoreType.DMA,
        pltpu.SemaphoreType.DMA,
    ],
)


@jax.jit
def kernel(token_ids, embed_table, pos_table):
    tok = token_ids.reshape(-1).astype(jnp.int32)
    out = _embed(tok, embed_table, pos_table)
    return out.reshape(_BATCH, _SEQ, _D)


# speedup vs baseline: 1.4769x; 1.4769x over previous
import functools

import jax
import jax.numpy as jnp
from jax import lax
from jax.experimental import pallas as pl
from jax.experimental.pallas import tpu as pltpu
from jax.experimental.pallas import tpu_sc as plsc

_NC = 2
_NS = 16
_NW = _NC * _NS
_L = 16

_BATCH = 4
_SEQ = 2048
_D = 1024
_S_PER_W = _SEQ // _NW
_CHUNK = 32
_NHALF = _S_PER_W // _CHUNK
_NCHUNK = _NHALF * _BATCH


def _body(tok_hbm, emb_hbm, pos_hbm, out_hbm,
          pbuf, ebuf0, ebuf1, idxv, gsem0, gsem1, ssem0, ssem1):
    wid = lax.axis_index("s") * _NC + lax.axis_index("c")
    s0 = wid * _S_PER_W

    for b in range(_BATCH):
        pltpu.sync_copy(tok_hbm.at[pl.ds(b * _SEQ + s0, _S_PER_W)],
                        idxv.at[pl.ds(b * _S_PER_W, _S_PER_W)])

    ebufs = (ebuf0, ebuf1)
    gsems = (gsem0, gsem1)
    ssems = (ssem0, ssem1)

    def idx_off(k):
        return (k % _BATCH) * _S_PER_W + (k // _BATCH) * _CHUNK

    def out_off(k):
        return (k % _BATCH) * _SEQ + s0 + (k // _BATCH) * _CHUNK

    gat = [None, None]
    scat = [None, None]

    pltpu.sync_copy(pos_hbm.at[pl.ds(s0, _CHUNK)], pbuf)
    gat[0] = pltpu.async_copy(
        emb_hbm.at[idxv.at[pl.ds(idx_off(0), _CHUNK)]], ebufs[0], gsems[0])

    for k in range(_NCHUNK):
        cur = k % 2
        nxt = 1 - cur
        if k + 1 < _NCHUNK:
            if scat[nxt] is not None:
                scat[nxt].wait()
            gat[nxt] = pltpu.async_copy(
                emb_hbm.at[idxv.at[pl.ds(idx_off(k + 1), _CHUNK)]],
                ebufs[nxt], gsems[nxt])
        gat[cur].wait()

        if k == _BATCH:
            pltpu.sync_copy(pos_hbm.at[pl.ds(s0 + _CHUNK, _CHUNK)], pbuf)

        ebuf = ebufs[cur]

        @plsc.parallel_loop(0, _CHUNK, 1)
        def row_add(r, _ebuf=ebuf):
            for j in range(_D // _L):
                sl = pl.ds(j * _L, _L)
                plsc.addupdate(_ebuf.at[r, sl], pbuf[r, sl])

        scat[cur] = pltpu.async_copy(
            ebuf, out_hbm.at[pl.ds(out_off(k), _CHUNK)], ssems[cur])

    scat[0].wait()
    scat[1].wait()


_mesh = plsc.VectorSubcoreMesh(core_axis_name="c", subcore_axis_name="s")

_embed = pl.kernel(
    _body,
    out_type=jax.ShapeDtypeStruct((_BATCH * _SEQ, _D), jnp.float32),
    mesh=_mesh,
    scratch_types=[
        pltpu.VMEM((_CHUNK, _D), jnp.float32),
        pltpu.VMEM((_CHUNK, _D), jnp.float32),
        pltpu.VMEM((_CHUNK, _D), jnp.float32),
        pltpu.VMEM((_BATCH * _S_PER_W,), jnp.int32),
        pltpu.SemaphoreType.DMA,
        pltpu.SemaphoreType.DMA,
        pltpu.SemaphoreType.DMA,
        pltpu.SemaphoreType.DMA,
    ],
)


@jax.jit
def kernel(token_ids, embed_table, pos_table):
    tok = token_ids.reshape(-1).astype(jnp.int32)
    out = _embed(tok, embed_table, pos_table)
    return out.reshape(_BATCH, _SEQ, _D)


# speedup vs baseline: 1.6008x; 1.0839x over previous
import functools

import jax
import jax.numpy as jnp
from jax import lax
from jax.experimental import pallas as pl
from jax.experimental.pallas import tpu as pltpu
from jax.experimental.pallas import tpu_sc as plsc

_NC = 2
_NS = 16
_NW = _NC * _NS
_L = 16

_BATCH = 4
_SEQ = 2048
_D = 1024
_S_PER_W = _SEQ // _NW
_HALF = 32
_CHUNK = 16
_NBUF = 4
_NCHUNK = _BATCH * _S_PER_W // _CHUNK
_PER_H = _NCHUNK // 2


def _body(tok_hbm, emb_hbm, pos_hbm, out_hbm,
          pbuf, ebuf0, ebuf1, ebuf2, ebuf3, idxv,
          gsem0, gsem1, gsem2, gsem3, ssem0, ssem1, ssem2, ssem3):
    wid = lax.axis_index("s") * _NC + lax.axis_index("c")
    s0 = wid * _S_PER_W

    for b in range(_BATCH):
        pltpu.sync_copy(tok_hbm.at[pl.ds(b * _SEQ + s0, _S_PER_W)],
                        idxv.at[pl.ds(b * _S_PER_W, _S_PER_W)])

    ebufs = (ebuf0, ebuf1, ebuf2, ebuf3)
    gsems = (gsem0, gsem1, gsem2, gsem3)
    ssems = (ssem0, ssem1, ssem2, ssem3)

    def _hbq(k):
        return k // _PER_H, (k // 2) % _BATCH, k % 2

    def idx_off(k):
        h, b, q = _hbq(k)
        return b * _S_PER_W + h * _HALF + q * _CHUNK

    def out_off(k):
        h, b, q = _hbq(k)
        return b * _SEQ + s0 + h * _HALF + q * _CHUNK

    gat = [None] * _NCHUNK
    sct = [None] * _NCHUNK

    def gfire(k):
        i = k % _NBUF
        if k >= _NBUF:
            sct[k - _NBUF].wait()
        gat[k] = pltpu.async_copy(
            emb_hbm.at[idxv.at[pl.ds(idx_off(k), _CHUNK)]],
            ebufs[i], gsems[i])

    pltpu.sync_copy(pos_hbm.at[pl.ds(s0, _HALF)], pbuf)
    for k in range(_NBUF - 1):
        gfire(k)

    for k in range(_NCHUNK):
        if k + _NBUF - 1 < _NCHUNK:
            gfire(k + _NBUF - 1)
        gat[k].wait()
        if k == _PER_H:
            pltpu.sync_copy(pos_hbm.at[pl.ds(s0 + _HALF, _HALF)], pbuf)

        i = k % _NBUF
        ebuf = ebufs[i]
        p0 = (k % 2) * _CHUNK

        @plsc.parallel_loop(0, _CHUNK, 1)
        def row_add(r, _ebuf=ebuf, _p0=p0):
            @plsc.parallel_loop(0, _D // _L, 8)
            def lane_add(j):
                for u in range(8):
                    sl = pl.ds((j + u) * _L, _L)
                    plsc.addupdate(_ebuf.at[r, sl], pbuf[_p0 + r, sl])

        sct[k] = pltpu.async_copy(
            ebuf, out_hbm.at[pl.ds(out_off(k), _CHUNK)], ssems[i])

    for k in range(_NCHUNK - _NBUF, _NCHUNK):
        sct[k].wait()


_mesh = plsc.VectorSubcoreMesh(core_axis_name="c", subcore_axis_name="s")

_embed = pl.kernel(
    _body,
    out_type=jax.ShapeDtypeStruct((_BATCH * _SEQ, _D), jnp.float32),
    mesh=_mesh,
    scratch_types=[
        pltpu.VMEM((_HALF, _D), jnp.float32),
        pltpu.VMEM((_CHUNK, _D), jnp.float32),
        pltpu.VMEM((_CHUNK, _D), jnp.float32),
        pltpu.VMEM((_CHUNK, _D), jnp.float32),
        pltpu.VMEM((_CHUNK, _D), jnp.float32),
        pltpu.VMEM((_BATCH * _S_PER_W,), jnp.int32),
        pltpu.SemaphoreType.DMA, pltpu.SemaphoreType.DMA,
        pltpu.SemaphoreType.DMA, pltpu.SemaphoreType.DMA,
        pltpu.SemaphoreType.DMA, pltpu.SemaphoreType.DMA,
        pltpu.SemaphoreType.DMA, pltpu.SemaphoreType.DMA,
    ],
)


@jax.jit
def kernel(token_ids, embed_table, pos_table):
    tok = token_ids.reshape(-1).astype(jnp.int32)
    out = _embed(tok, embed_table, pos_table)
    return out.reshape(_BATCH, _SEQ, _D)


# speedup vs baseline: 2.0416x; 1.2753x over previous
import functools

import jax
import jax.numpy as jnp
from jax import lax
from jax.experimental import pallas as pl
from jax.experimental.pallas import tpu as pltpu
from jax.experimental.pallas import tpu_sc as plsc

_NC = 2
_NS = 16
_NW = _NC * _NS
_L = 16

_BATCH = 4
_SEQ = 2048
_D = 1024
_S_PER_W = _SEQ // _NW
_SEG = 8
_NCHUNK = _S_PER_W // _SEG
_ROWS = _BATCH * _SEG
_NBUF = 3


def _body(tok_hbm, emb_hbm, pos_hbm, out_hbm,
          ebuf0, ebuf1, ebuf2, pbuf0, pbuf1, idxv,
          gsem0, gsem1, gsem2, ssem0, ssem1, ssem2, psem0, psem1):
    wid = lax.axis_index("s") * _NC + lax.axis_index("c")
    s0 = wid * _S_PER_W

    for b in range(_BATCH):
        pltpu.sync_copy(tok_hbm.at[pl.ds(b * _SEQ + s0, _S_PER_W)],
                        idxv.at[pl.ds(b * _S_PER_W, _S_PER_W)])

    ebufs = (ebuf0, ebuf1, ebuf2)
    pbufs = (pbuf0, pbuf1)
    gsems = (gsem0, gsem1, gsem2)
    ssems = (ssem0, ssem1, ssem2)
    psems = (psem0, psem1)

    gat = [None] * _NCHUNK
    pf = [None] * _NCHUNK
    sct = [None] * _NCHUNK

    def gfire(k):
        i = k % _NBUF
        if k >= _NBUF:
            for s in sct[k - _NBUF]:
                s.wait()
        gat[k] = [
            pltpu.async_copy(
                emb_hbm.at[idxv.at[pl.ds(b * _S_PER_W + k * _SEG, _SEG)]],
                ebufs[i].at[pl.ds(b * _SEG, _SEG)], gsems[i])
            for b in range(_BATCH)
        ]

    def pfire(k):
        pf[k] = pltpu.async_copy(
            pos_hbm.at[pl.ds(s0 + k * _SEG, _SEG)], pbufs[k % 2], psems[k % 2])

    pfire(0)
    gfire(0)
    pfire(1)
    gfire(1)
    for k in range(_NCHUNK):
        if k + 2 < _NCHUNK:
            gfire(k + 2)
        for g in gat[k]:
            g.wait()
        pf[k].wait()

        i = k % _NBUF
        ebuf = ebufs[i]
        pbuf = pbufs[k % 2]

        @plsc.parallel_loop(0, _SEG, 1)
        def row_add(r, _ebuf=ebuf, _pbuf=pbuf):
            @plsc.parallel_loop(0, _D // _L, 2)
            def lane_add(j):
                for u in range(2):
                    sl = pl.ds((j + u) * _L, _L)
                    x = _pbuf[r, sl]
                    for b in range(_BATCH):
                        plsc.addupdate(_ebuf.at[b * _SEG + r, sl], x)

        if k + 2 < _NCHUNK:
            pfire(k + 2)

        sct[k] = [
            pltpu.async_copy(
                ebuf.at[pl.ds(b * _SEG, _SEG)],
                out_hbm.at[pl.ds(b * _SEQ + s0 + k * _SEG, _SEG)], ssems[i])
            for b in range(_BATCH)
        ]

    for k in range(_NCHUNK - _NBUF, _NCHUNK):
        for s in sct[k]:
            s.wait()


_mesh = plsc.VectorSubcoreMesh(core_axis_name="c", subcore_axis_name="s")

_embed = pl.kernel(
    _body,
    out_type=jax.ShapeDtypeStruct((_BATCH * _SEQ, _D), jnp.float32),
    mesh=_mesh,
    scratch_types=[
        pltpu.VMEM((_ROWS, _D), jnp.float32),
        pltpu.VMEM((_ROWS, _D), jnp.float32),
        pltpu.VMEM((_ROWS, _D), jnp.float32),
        pltpu.VMEM((_SEG, _D), jnp.float32),
        pltpu.VMEM((_SEG, _D), jnp.float32),
        pltpu.VMEM((_BATCH * _S_PER_W,), jnp.int32),
        pltpu.SemaphoreType.DMA, pltpu.SemaphoreType.DMA,
        pltpu.SemaphoreType.DMA, pltpu.SemaphoreType.DMA,
        pltpu.SemaphoreType.DMA, pltpu.SemaphoreType.DMA,
        pltpu.SemaphoreType.DMA, pltpu.SemaphoreType.DMA,
    ],
)


@jax.jit
def kernel(token_ids, embed_table, pos_table):
    tok = token_ids.reshape(-1).astype(jnp.int32)
    out = _embed(tok, embed_table, pos_table)
    return out.reshape(_BATCH, _SEQ, _D)


# speedup vs baseline: 2.0546x; 1.0064x over previous
import functools

import jax
import jax.numpy as jnp
from jax import lax
from jax.experimental import pallas as pl
from jax.experimental.pallas import tpu as pltpu
from jax.experimental.pallas import tpu_sc as plsc

_NC = 2
_NS = 16
_NW = _NC * _NS
_L = 16

_BATCH = 4
_SEQ = 2048
_D = 1024
_S_PER_W = _SEQ // _NW
_SEG = 8
_NCHUNK = _S_PER_W // _SEG
_ROWS = _BATCH * _SEG
_NBUF = 3


def _body(tok_hbm, emb_hbm, pos_hbm, out_hbm,
          ebuf0, ebuf1, ebuf2, pbuf0, pbuf1, idxv,
          gsem0, gsem1, gsem2, ssem0, ssem1, ssem2, psem0, psem1):
    wid = lax.axis_index("s") * _NC + lax.axis_index("c")
    s0 = wid * _S_PER_W

    for b in range(_BATCH):
        pltpu.sync_copy(tok_hbm.at[b, pl.ds(s0, _S_PER_W)],
                        idxv.at[pl.ds(b * _S_PER_W, _S_PER_W)])

    ebufs = (ebuf0, ebuf1, ebuf2)
    pbufs = (pbuf0, pbuf1)
    gsems = (gsem0, gsem1, gsem2)
    ssems = (ssem0, ssem1, ssem2)
    psems = (psem0, psem1)

    gat = [None] * _NCHUNK
    pf = [None] * _NCHUNK
    sct = [None] * _NCHUNK

    def gfire(k):
        i = k % _NBUF
        if k >= _NBUF:
            for s in sct[k - _NBUF]:
                s.wait()
        gat[k] = [
            pltpu.async_copy(
                emb_hbm.at[idxv.at[pl.ds(b * _S_PER_W + k * _SEG, _SEG)]],
                ebufs[i].at[pl.ds(b * _SEG, _SEG)], gsems[i])
            for b in range(_BATCH)
        ]

    def pfire(k):
        pf[k] = pltpu.async_copy(
            pos_hbm.at[pl.ds(s0 + k * _SEG, _SEG)], pbufs[k % 2], psems[k % 2])

    pfire(0)
    gfire(0)
    pfire(1)
    gfire(1)
    for k in range(_NCHUNK):
        if k + 2 < _NCHUNK:
            gfire(k + 2)
        for g in gat[k]:
            g.wait()
        pf[k].wait()

        i = k % _NBUF
        ebuf = ebufs[i]
        pbuf = pbufs[k % 2]

        @plsc.parallel_loop(0, _SEG, 1)
        def row_add(r, _ebuf=ebuf, _pbuf=pbuf):
            @plsc.parallel_loop(0, _D // _L, 2)
            def lane_add(j):
                for u in range(2):
                    sl = pl.ds((j + u) * _L, _L)
                    x = _pbuf[r, sl]
                    for b in range(_BATCH):
                        plsc.addupdate(_ebuf.at[b * _SEG + r, sl], x)

        if k + 2 < _NCHUNK:
            pfire(k + 2)

        sct[k] = [
            pltpu.async_copy(
                ebuf.at[pl.ds(b * _SEG, _SEG)],
                out_hbm.at[b, pl.ds(s0 + k * _SEG, _SEG)], ssems[i])
            for b in range(_BATCH)
        ]

    for k in range(_NCHUNK - _NBUF, _NCHUNK):
        for s in sct[k]:
            s.wait()


_mesh = plsc.VectorSubcoreMesh(core_axis_name="c", subcore_axis_name="s")

_embed = pl.kernel(
    _body,
    out_type=jax.ShapeDtypeStruct((_BATCH, _SEQ, _D), jnp.float32),
    mesh=_mesh,
    scratch_types=[
        pltpu.VMEM((_ROWS, _D), jnp.float32),
        pltpu.VMEM((_ROWS, _D), jnp.float32),
        pltpu.VMEM((_ROWS, _D), jnp.float32),
        pltpu.VMEM((_SEG, _D), jnp.float32),
        pltpu.VMEM((_SEG, _D), jnp.float32),
        pltpu.VMEM((_BATCH * _S_PER_W,), jnp.int32),
        pltpu.SemaphoreType.DMA, pltpu.SemaphoreType.DMA,
        pltpu.SemaphoreType.DMA, pltpu.SemaphoreType.DMA,
        pltpu.SemaphoreType.DMA, pltpu.SemaphoreType.DMA,
        pltpu.SemaphoreType.DMA, pltpu.SemaphoreType.DMA,
    ],
)


@jax.jit
def kernel(token_ids, embed_table, pos_table):
    return _embed(token_ids, embed_table, pos_table)
